# Initial kernel scaffold; baseline (speedup 1.0000x reference)
import jax
import jax.numpy as jnp
from jax.experimental import pallas as pl


def kernel(idx, codebook):
    raise NotImplementedError("write your pallas kernel here")


# speedup vs baseline: 5.7579x; 5.7579x over previous
import functools

import jax
import jax.numpy as jnp
from jax import lax
from jax.experimental import pallas as pl
from jax.experimental.pallas import tpu as pltpu
from jax.experimental.pallas import tpu_sc as plsc

_NUM_CORES = 2
_NUM_SUBCORES = 16
_NW = _NUM_CORES * _NUM_SUBCORES
_LANES = 16

_B, _T = 16384, 200
_C, _D = 256, 8
_N = _B * _T
_PER_W = _N // _NW
_CHUNK = 10240
_NCHUNK = _PER_W // _CHUNK
_GROUPS = _CHUNK // _LANES


def _make_lookup():
    mesh = plsc.VectorSubcoreMesh(core_axis_name="c", subcore_axis_name="s")

    @functools.partial(
        pl.kernel,
        out_type=jax.ShapeDtypeStruct((_N * _D,), jnp.float32),
        mesh=mesh,
        scratch_types=[
            pltpu.VMEM((_C * _D,), jnp.float32),
            pltpu.VMEM((_CHUNK,), jnp.int32),
            pltpu.VMEM((_CHUNK * _D,), jnp.float32),
        ],
        compiler_params=pltpu.CompilerParams(needs_layout_passes=False),
    )
    def lookup(idx_hbm, table_hbm, out_hbm, table_v, idx_v, out_v):
        wid = lax.axis_index("s") * _NUM_CORES + lax.axis_index("c")
        pltpu.sync_copy(table_hbm, table_v)
        lane = lax.iota(jnp.int32, _LANES)
        pos0 = lane * _D
        for c in range(_NCHUNK):
            base = wid * _PER_W + c * _CHUNK
            pltpu.sync_copy(idx_hbm.at[pl.ds(base, _CHUNK)], idx_v)

            @pl.loop(0, _GROUPS)
            def _group(g):
                i16 = idx_v[pl.ds(g * _LANES, _LANES)]
                gidx0 = i16 * _D
                obase = pos0 + g * (_LANES * _D)
                for j in range(_D):
                    vals = plsc.load_gather(table_v, [gidx0 + j])
                    plsc.store_scatter(out_v, [obase + j], vals)

            pltpu.sync_copy(out_v, out_hbm.at[pl.ds(base * _D, _CHUNK * _D)])

    return lookup


_lookup = _make_lookup()


def kernel(idx, codebook):
    b, t = idx.shape
    _, d = codebook.shape
    out = _lookup(idx.reshape(-1), codebook.reshape(-1))
    return out.reshape(b, t, d)
